# baseline (device time: 13233 ns/iter reference)
import jax
import jax.numpy as jnp
from jax import lax
from jax.experimental import pallas as pl
from jax.experimental.pallas import tpu as pltpu

N_DEV = 8
G = 8
NB = 2
NR = 4


def kernel(x):
    m, n = x.shape
    W = n // NB
    C = m // NR

    def body(x_hbm, out_ref, vbuf, comm_ref, copy_sems, send_sems, recv_sems):
        my = lax.axis_index("i")

        barrier_sem = pltpu.get_barrier_semaphore()
        for k in range(1, N_DEV):
            dst = lax.rem(my + k, N_DEV)
            pl.semaphore_signal(
                barrier_sem, inc=1,
                device_id=(dst,), device_id_type=pl.DeviceIdType.MESH,
            )

        copies = {}
        for cb in range(NB):
            for rc in range(NR):
                cp = pltpu.make_async_copy(
                    x_hbm.at[pl.ds(rc * C, C), pl.ds(cb * W, W)],
                    vbuf.at[cb, rc],
                    copy_sems.at[cb, rc],
                )
                cp.start()
                copies[cb, rc] = cp

        rdmas = []
        for cb in range(NB):
            acc = jnp.zeros((G, W), jnp.float32)
            for rc in range(NR):
                copies[cb, rc].wait()
                acc = acc + jnp.sum(vbuf[cb, rc].reshape(G, C // G, W), axis=1)
            comm_ref[my, cb, :, :] = acc

            if cb == 0:
                pl.semaphore_wait(barrier_sem, N_DEV - 1)

            for k in range(1, N_DEV):
                dst = lax.rem(my + k, N_DEV)
                rdma = pltpu.make_async_remote_copy(
                    src_ref=comm_ref.at[my, cb],
                    dst_ref=comm_ref.at[my, cb],
                    send_sem=send_sems.at[k, cb],
                    recv_sem=recv_sems.at[k, cb],
                    device_id=(dst,),
                    device_id_type=pl.DeviceIdType.MESH,
                )
                rdma.start()
                rdmas.append(rdma)

        for cb in range(NB):
            for k in range(1, N_DEV):
                src = lax.rem(my - k + N_DEV, N_DEV)
                recv = pltpu.make_async_remote_copy(
                    src_ref=comm_ref.at[src, cb],
                    dst_ref=comm_ref.at[src, cb],
                    send_sem=send_sems.at[k, cb],
                    recv_sem=recv_sems.at[k, cb],
                    device_id=(src,),
                    device_id_type=pl.DeviceIdType.MESH,
                )
                recv.wait_recv()

        for rdma in rdmas:
            rdma.wait_send()

        for cb in range(NB):
            out_ref[:, pl.ds(cb * W, W)] = jnp.sum(
                comm_ref[:, cb].reshape(N_DEV * G, W), axis=0, keepdims=True
            )

    return pl.pallas_call(
        body,
        out_shape=jax.ShapeDtypeStruct((1, n), jnp.float32),
        in_specs=[pl.BlockSpec(memory_space=pl.ANY)],
        out_specs=pl.BlockSpec(memory_space=pltpu.VMEM),
        scratch_shapes=[
            pltpu.VMEM((NB, NR, C, W), jnp.float32),
            pltpu.VMEM((N_DEV, NB, G, W), jnp.float32),
            pltpu.SemaphoreType.DMA((NB, NR)),
            pltpu.SemaphoreType.DMA((N_DEV, NB)),
            pltpu.SemaphoreType.DMA((N_DEV, NB)),
        ],
        compiler_params=pltpu.CompilerParams(collective_id=0),
    )(x)
